# baseline (speedup 1.0000x reference)
import jax
import jax.numpy as jnp
from jax import lax
from jax.experimental import pallas as pl
from jax.experimental.pallas import tpu as pltpu
from jax.experimental.pallas import tpu_sc as plsc

STATE_RANK = 128
MEASURE_RANK = 64
BATCH = STATE_RANK * MEASURE_RANK
PAD = 128

NUM_CORES = 2

BB = 256
NB = BATCH // BB


def _sc_scatter_body(pred_hbm, tile_hbm, buf, sem):
    wid = lax.axis_index("s") * NUM_CORES + lax.axis_index("c")

    @pl.when(wid == 0)
    def _():
        pltpu.sync_copy(pred_hbm, buf)
        pltpu.sync_copy(buf, tile_hbm)


@jax.jit
def _innovation_tile_sc(pred2d):
    mesh = plsc.VectorSubcoreMesh(core_axis_name="c", subcore_axis_name="s")
    return pl.kernel(
        _sc_scatter_body,
        out_type=jax.ShapeDtypeStruct((STATE_RANK, MEASURE_RANK), jnp.float32),
        mesh=mesh,
        scratch_types=[
            pltpu.VMEM((STATE_RANK, MEASURE_RANK), jnp.float32),
            pltpu.SemaphoreType.DMA,
        ],
    )(pred2d)


def _tc_broadcast_body(tile_ref, out_ref, scratch, sem):
    scratch[:, :, :MEASURE_RANK] = jnp.broadcast_to(
        tile_ref[...][None], (BB, STATE_RANK, MEASURE_RANK)
    )
    copies = [
        pltpu.make_async_copy(scratch, out_ref.at[pl.ds(j * BB, BB)], sem)
        for j in range(NB)
    ]
    for c in copies:
        c.start()
    for c in copies:
        c.wait()


@jax.jit
def _broadcast_tc(tile):
    padded = pl.pallas_call(
        _tc_broadcast_body,
        in_specs=[pl.BlockSpec(memory_space=pltpu.VMEM)],
        out_specs=pl.BlockSpec(memory_space=pl.ANY),
        out_shape=jax.ShapeDtypeStruct((BATCH, STATE_RANK, PAD), jnp.float32),
        scratch_shapes=[
            pltpu.VMEM((BB, STATE_RANK, PAD), jnp.float32),
            pltpu.SemaphoreType.DMA,
        ],
    )(tile)
    return padded[:, :, :MEASURE_RANK]


def kernel(input, unconstrained_params):
    del input
    pred2d = unconstrained_params.reshape(STATE_RANK, MEASURE_RANK)
    tile = _innovation_tile_sc(pred2d)
    return _broadcast_tc(tile)


# speedup vs baseline: 3.5048x; 3.5048x over previous
import jax
import jax.numpy as jnp
from jax import lax
from jax.experimental import pallas as pl
from jax.experimental.pallas import tpu as pltpu
from jax.experimental.pallas import tpu_sc as plsc

STATE_RANK = 128
MEASURE_RANK = 64
BATCH = STATE_RANK * MEASURE_RANK
PARAMS = STATE_RANK * MEASURE_RANK

NUM_CORES = 2
NUM_SUBCORES = 16
NUM_WORKERS = NUM_CORES * NUM_SUBCORES
WORDS_PER_WORKER = PARAMS // NUM_WORKERS
GATHER = 128

BB = 256
NB = BATCH // BB


def _sc_scatter_body(pred_hbm, idx_hbm, tilet_hbm, idx_v, row_v, sem):
    wid = lax.axis_index("s") * NUM_CORES + lax.axis_index("c")
    base = wid * WORDS_PER_WORKER
    for j in range(WORDS_PER_WORKER // GATHER):
        off = base + j * GATHER
        pltpu.sync_copy(idx_hbm.at[pl.ds(off, GATHER)], idx_v)
        pltpu.async_copy(pred_hbm.at[idx_v], row_v, sem).wait()
        pltpu.sync_copy(row_v, tilet_hbm.at[pl.ds(off, GATHER)])


@jax.jit
def _innovation_tile_t_sc(pred, idx):
    mesh = plsc.VectorSubcoreMesh(core_axis_name="c", subcore_axis_name="s")
    return pl.kernel(
        _sc_scatter_body,
        out_type=jax.ShapeDtypeStruct((PARAMS,), jnp.float32),
        mesh=mesh,
        scratch_types=[
            pltpu.VMEM((GATHER,), jnp.int32),
            pltpu.VMEM((GATHER,), jnp.float32),
            pltpu.SemaphoreType.DMA,
        ],
    )(pred, idx)


def _tc_broadcast_body(tilet_ref, out_ref, scratch, sem):
    scratch[...] = jnp.broadcast_to(
        tilet_ref[...][None], (BB, MEASURE_RANK, STATE_RANK)
    )
    copies = [
        pltpu.make_async_copy(scratch, out_ref.at[pl.ds(j * BB, BB)], sem)
        for j in range(NB)
    ]
    for c in copies:
        c.start()
    for c in copies:
        c.wait()


@jax.jit
def _broadcast_tc(tilet):
    outt = pl.pallas_call(
        _tc_broadcast_body,
        in_specs=[pl.BlockSpec(memory_space=pltpu.VMEM)],
        out_specs=pl.BlockSpec(memory_space=pl.ANY),
        out_shape=jax.ShapeDtypeStruct(
            (BATCH, MEASURE_RANK, STATE_RANK), jnp.float32
        ),
        scratch_shapes=[
            pltpu.VMEM((BB, MEASURE_RANK, STATE_RANK), jnp.float32),
            pltpu.SemaphoreType.DMA,
        ],
    )(tilet)
    return jnp.transpose(outt, (0, 2, 1))


def kernel(input, unconstrained_params):
    del input
    p = jnp.arange(PARAMS, dtype=jnp.int32)
    idx = (p % STATE_RANK) * MEASURE_RANK + p // STATE_RANK
    tilet = _innovation_tile_t_sc(unconstrained_params, idx).reshape(
        MEASURE_RANK, STATE_RANK
    )
    return _broadcast_tc(tilet)


# speedup vs baseline: 3.5275x; 1.0065x over previous
import jax
import jax.numpy as jnp
from jax import lax
from jax.experimental import pallas as pl
from jax.experimental.pallas import tpu as pltpu
from jax.experimental.pallas import tpu_sc as plsc

STATE_RANK = 128
MEASURE_RANK = 64
BATCH = STATE_RANK * MEASURE_RANK
PARAMS = STATE_RANK * MEASURE_RANK

NUM_CORES = 2
NUM_SUBCORES = 16
NUM_WORKERS = NUM_CORES * NUM_SUBCORES
WORDS_PER_WORKER = PARAMS // NUM_WORKERS
GATHER = 128

BB = 256
NB = BATCH // BB


def _sc_scatter_body(pred_hbm, idx_hbm, tilet_hbm, idx_v, row_v, sem):
    wid = lax.axis_index("s") * NUM_CORES + lax.axis_index("c")
    base = wid * WORDS_PER_WORKER
    pltpu.sync_copy(idx_hbm.at[pl.ds(base, WORDS_PER_WORKER)], idx_v)
    copies = [
        pltpu.async_copy(
            pred_hbm.at[idx_v.at[pl.ds(j * GATHER, GATHER)]],
            row_v.at[pl.ds(j * GATHER, GATHER)],
            sem,
        )
        for j in range(WORDS_PER_WORKER // GATHER)
    ]
    for c in copies:
        c.wait()
    pltpu.sync_copy(row_v, tilet_hbm.at[pl.ds(base, WORDS_PER_WORKER)])


@jax.jit
def _innovation_tile_t_sc(pred, idx):
    mesh = plsc.VectorSubcoreMesh(core_axis_name="c", subcore_axis_name="s")
    return pl.kernel(
        _sc_scatter_body,
        out_type=jax.ShapeDtypeStruct((PARAMS,), jnp.float32),
        mesh=mesh,
        scratch_types=[
            pltpu.VMEM((WORDS_PER_WORKER,), jnp.int32),
            pltpu.VMEM((WORDS_PER_WORKER,), jnp.float32),
            pltpu.SemaphoreType.DMA,
        ],
    )(pred, idx)


def _tc_broadcast_body(tilet_ref, out_ref, scratch, sem):
    scratch[...] = jnp.broadcast_to(
        tilet_ref[...][None], (BB, MEASURE_RANK, STATE_RANK)
    )
    copies = [
        pltpu.make_async_copy(scratch, out_ref.at[pl.ds(j * BB, BB)], sem)
        for j in range(NB)
    ]
    for c in copies:
        c.start()
    for c in copies:
        c.wait()


@jax.jit
def _broadcast_tc(tilet):
    outt = pl.pallas_call(
        _tc_broadcast_body,
        in_specs=[pl.BlockSpec(memory_space=pltpu.VMEM)],
        out_specs=pl.BlockSpec(memory_space=pl.ANY),
        out_shape=jax.ShapeDtypeStruct(
            (BATCH, MEASURE_RANK, STATE_RANK), jnp.float32
        ),
        scratch_shapes=[
            pltpu.VMEM((BB, MEASURE_RANK, STATE_RANK), jnp.float32),
            pltpu.SemaphoreType.DMA,
        ],
    )(tilet)
    return jnp.transpose(outt, (0, 2, 1))


def kernel(input, unconstrained_params):
    del input
    p = jnp.arange(PARAMS, dtype=jnp.int32)
    idx = (p % STATE_RANK) * MEASURE_RANK + p // STATE_RANK
    tilet = _innovation_tile_t_sc(unconstrained_params, idx).reshape(
        MEASURE_RANK, STATE_RANK
    )
    return _broadcast_tc(tilet)


# speedup vs baseline: 3.5385x; 1.0031x over previous
import jax
import jax.numpy as jnp
from jax import lax
from jax.experimental import pallas as pl
from jax.experimental.pallas import tpu as pltpu
from jax.experimental.pallas import tpu_sc as plsc

STATE_RANK = 128
MEASURE_RANK = 64
BATCH = STATE_RANK * MEASURE_RANK
PARAMS = STATE_RANK * MEASURE_RANK

NUM_CORES = 2
NUM_SUBCORES = 16
NUM_WORKERS = NUM_CORES * NUM_SUBCORES
WORDS_PER_WORKER = PARAMS // NUM_WORKERS
GATHER = 128

BB = 256
NB = BATCH // BB


def _sc_scatter_body(pred_hbm, idx_hbm, tilet_hbm, idx_v, row_v, sem):
    wid = lax.axis_index("s") * NUM_CORES + lax.axis_index("c")
    base = wid * WORDS_PER_WORKER
    pltpu.sync_copy(idx_hbm.at[pl.ds(base, WORDS_PER_WORKER)], idx_v)
    copies = [
        pltpu.async_copy(
            pred_hbm.at[idx_v.at[pl.ds(j * GATHER, GATHER)]],
            row_v.at[pl.ds(j * GATHER, GATHER)],
            sem,
        )
        for j in range(WORDS_PER_WORKER // GATHER)
    ]
    for c in copies:
        c.wait()
    pltpu.sync_copy(row_v, tilet_hbm.at[pl.ds(base, WORDS_PER_WORKER)])


@jax.jit
def _innovation_tile_t_sc(pred, idx):
    mesh = plsc.VectorSubcoreMesh(core_axis_name="c", subcore_axis_name="s")
    return pl.kernel(
        _sc_scatter_body,
        out_type=jax.ShapeDtypeStruct((PARAMS,), jnp.float32),
        mesh=mesh,
        scratch_types=[
            pltpu.VMEM((WORDS_PER_WORKER,), jnp.int32),
            pltpu.VMEM((WORDS_PER_WORKER,), jnp.float32),
            pltpu.SemaphoreType.DMA,
        ],
        compiler_params=pltpu.CompilerParams(skip_device_barrier=True),
    )(pred, idx)


def _tc_broadcast_body(tilet_ref, out_ref, scratch, sem):
    scratch[...] = jnp.broadcast_to(
        tilet_ref[...][None], (BB, MEASURE_RANK, STATE_RANK)
    )
    copies = [
        pltpu.make_async_copy(scratch, out_ref.at[pl.ds(j * BB, BB)], sem)
        for j in range(NB)
    ]
    for c in copies:
        c.start()
    for c in copies:
        c.wait()


@jax.jit
def _broadcast_tc(tilet):
    outt = pl.pallas_call(
        _tc_broadcast_body,
        in_specs=[pl.BlockSpec(memory_space=pltpu.VMEM)],
        out_specs=pl.BlockSpec(memory_space=pl.ANY),
        out_shape=jax.ShapeDtypeStruct(
            (BATCH, MEASURE_RANK, STATE_RANK), jnp.float32
        ),
        scratch_shapes=[
            pltpu.VMEM((BB, MEASURE_RANK, STATE_RANK), jnp.float32),
            pltpu.SemaphoreType.DMA,
        ],
    )(tilet)
    return jnp.transpose(outt, (0, 2, 1))


def kernel(input, unconstrained_params):
    del input
    p = jnp.arange(PARAMS, dtype=jnp.int32)
    idx = (p % STATE_RANK) * MEASURE_RANK + p // STATE_RANK
    tilet = _innovation_tile_t_sc(unconstrained_params, idx).reshape(
        MEASURE_RANK, STATE_RANK
    )
    return _broadcast_tc(tilet)


# speedup vs baseline: 3.5590x; 1.0058x over previous
import jax
import jax.numpy as jnp
from jax import lax
from jax.experimental import pallas as pl
from jax.experimental.pallas import tpu as pltpu
from jax.experimental.pallas import tpu_sc as plsc

STATE_RANK = 128
MEASURE_RANK = 64
BATCH = STATE_RANK * MEASURE_RANK
PARAMS = STATE_RANK * MEASURE_RANK

NUM_CORES = 2
NUM_SUBCORES = 16
NUM_WORKERS = NUM_CORES * NUM_SUBCORES
WORDS_PER_WORKER = PARAMS // NUM_WORKERS
GATHER = 128

BB = 128
NB = BATCH // BB


def _sc_scatter_body(pred_hbm, idx_hbm, tilet_hbm, idx_v, row_v, sem):
    wid = lax.axis_index("s") * NUM_CORES + lax.axis_index("c")
    base = wid * WORDS_PER_WORKER
    pltpu.sync_copy(idx_hbm.at[pl.ds(base, WORDS_PER_WORKER)], idx_v)
    copies = [
        pltpu.async_copy(
            pred_hbm.at[idx_v.at[pl.ds(j * GATHER, GATHER)]],
            row_v.at[pl.ds(j * GATHER, GATHER)],
            sem,
        )
        for j in range(WORDS_PER_WORKER // GATHER)
    ]
    for c in copies:
        c.wait()
    pltpu.sync_copy(row_v, tilet_hbm.at[pl.ds(base, WORDS_PER_WORKER)])


@jax.jit
def _innovation_tile_t_sc(pred, idx):
    mesh = plsc.VectorSubcoreMesh(core_axis_name="c", subcore_axis_name="s")
    return pl.kernel(
        _sc_scatter_body,
        out_type=jax.ShapeDtypeStruct((PARAMS,), jnp.float32),
        mesh=mesh,
        scratch_types=[
            pltpu.VMEM((WORDS_PER_WORKER,), jnp.int32),
            pltpu.VMEM((WORDS_PER_WORKER,), jnp.float32),
            pltpu.SemaphoreType.DMA,
        ],
    )(pred, idx)


def _tc_broadcast_body(tilet_ref, out_ref, scratch, sem):
    scratch[...] = jnp.broadcast_to(
        tilet_ref[...][None], (BB, MEASURE_RANK, STATE_RANK)
    )
    copies = [
        pltpu.make_async_copy(scratch, out_ref.at[pl.ds(j * BB, BB)], sem)
        for j in range(NB)
    ]
    for c in copies:
        c.start()
    for c in copies:
        c.wait()


@jax.jit
def _broadcast_tc(tilet):
    outt = pl.pallas_call(
        _tc_broadcast_body,
        in_specs=[pl.BlockSpec(memory_space=pltpu.VMEM)],
        out_specs=pl.BlockSpec(memory_space=pl.ANY),
        out_shape=jax.ShapeDtypeStruct(
            (BATCH, MEASURE_RANK, STATE_RANK), jnp.float32
        ),
        scratch_shapes=[
            pltpu.VMEM((BB, MEASURE_RANK, STATE_RANK), jnp.float32),
            pltpu.SemaphoreType.DMA,
        ],
    )(tilet)
    return jnp.transpose(outt, (0, 2, 1))


def kernel(input, unconstrained_params):
    del input
    p = jnp.arange(PARAMS, dtype=jnp.int32)
    idx = (p % STATE_RANK) * MEASURE_RANK + p // STATE_RANK
    tilet = _innovation_tile_t_sc(unconstrained_params, idx).reshape(
        MEASURE_RANK, STATE_RANK
    )
    return _broadcast_tc(tilet)
